# baseline (speedup 1.0000x reference)
import functools

import jax
import jax.numpy as jnp
from jax import lax
from jax.experimental import pallas as pl
from jax.experimental.pallas import tpu as pltpu
from jax.experimental.pallas import tpu_sc as plsc

MAXLEN = 500
EMBED_DIM = 64
BATCH = 4096

NC = 2
NS = 16
NW = NC * NS
ROWS_PER_W = BATCH // NW


def _make_sc_kernel():
  mesh = plsc.VectorSubcoreMesh(core_axis_name="c", subcore_axis_name="s")

  @functools.partial(
      pl.kernel,
      mesh=mesh,
      out_type=jax.ShapeDtypeStruct((BATCH, MAXLEN, EMBED_DIM), jnp.float32),
      compiler_params=pltpu.CompilerParams(use_tc_tiling_on_sc=False),
      scratch_types=[
          pltpu.VMEM((MAXLEN,), jnp.int32),
          pltpu.VMEM((MAXLEN,), jnp.int32),
          pltpu.VMEM((MAXLEN, EMBED_DIM), jnp.float32),
          pltpu.VMEM((MAXLEN, EMBED_DIM), jnp.float32),
          pltpu.VMEM((MAXLEN, EMBED_DIM), jnp.float32),
          pltpu.SemaphoreType.DMA,
          pltpu.SemaphoreType.DMA,
          pltpu.SemaphoreType.DMA,
          pltpu.SemaphoreType.DMA,
      ],
  )
  def k(x_hbm, tab_hbm, pos_hbm, out_hbm,
        idx0, idx1, rows0, rows1, pos_v, g0, g1, o0, o1):
    wid = lax.axis_index("s") * NC + lax.axis_index("c")
    row0 = wid * ROWS_PER_W

    idx = (idx0, idx1)
    rows = (rows0, rows1)
    gsem = (g0, g1)
    osem = (o0, o1)

    pltpu.sync_copy(pos_hbm, pos_v)

    pltpu.sync_copy(x_hbm.at[row0], idx0)
    pltpu.async_copy(tab_hbm.at[idx0], rows0, g0)

    def step(r, b, nb):
      pltpu.make_async_copy(tab_hbm.at[idx[b]], rows[b], gsem[b]).wait()

      @pl.when(r < ROWS_PER_W - 1)
      def _():
        @pl.when(r >= 1)
        def _():
          pltpu.make_async_copy(
              rows[nb], out_hbm.at[row0 + r - 1], osem[nb]).wait()
        pltpu.sync_copy(x_hbm.at[row0 + r + 1], idx[nb])
        pltpu.async_copy(tab_hbm.at[idx[nb]], rows[nb], gsem[nb])

      rv = rows[b]

      def add_body(l, carry):
        for c in range(EMBED_DIM // 16):
          s = pl.ds(c * 16, 16)
          rv[l, s] = rv[l, s] + pos_v[l, s]
        return carry

      lax.fori_loop(0, MAXLEN, add_body, 0, unroll=4)

      pltpu.async_copy(rows[b], out_hbm.at[row0 + r], osem[b])

    def pair(g, carry):
      step(2 * g, 0, 1)
      step(2 * g + 1, 1, 0)
      return carry

    lax.fori_loop(0, ROWS_PER_W // 2, pair, 0)

    pltpu.make_async_copy(rows0, out_hbm.at[row0 + ROWS_PER_W - 2], o0).wait()
    pltpu.make_async_copy(rows1, out_hbm.at[row0 + ROWS_PER_W - 1], o1).wait()

  return k


_sc_kernel = _make_sc_kernel()


def kernel(x, token_table, pos_table):
  return _sc_kernel(x.astype(jnp.int32), token_table, pos_table)
